# Initial kernel scaffold; baseline (speedup 1.0000x reference)
import jax
import jax.numpy as jnp
from jax.experimental import pallas as pl


def kernel(x, edge_index, edge_weight, W_enc, W_b1, W_b2, W_dec, beta, gamma):
    raise NotImplementedError("write your pallas kernel here")


# speedup vs baseline: 3.0524x; 3.0524x over previous
import functools

import jax
import jax.numpy as jnp
from jax import lax
from jax.experimental import pallas as pl
from jax.experimental.pallas import tpu as pltpu
from jax.experimental.pallas import tpu_sc as plsc

N_PAD = 10240
RT = N_PAD // 16
K = 128
NW = 32
F = 128
ITERS = 10

_mesh = plsc.VectorSubcoreMesh(core_axis_name="c", subcore_axis_name="s")


def _make_scatter_step(C):
    @functools.partial(
        pl.kernel,
        mesh=_mesh,
        out_type=jax.ShapeDtypeStruct((2, N_PAD, F), jnp.float32),
        scratch_types=[
            pltpu.VMEM((K, F), jnp.float32),
            pltpu.VMEM((K,), jnp.int32),
            pltpu.VMEM((K,), jnp.int32),
            pltpu.VMEM((K, 16), jnp.float32),
            pltpu.VMEM_SHARED((N_PAD, F), jnp.float32),
            pltpu.SemaphoreType.DMA,
        ],
    )
    def scatter_step(uhalf_hbm, src_hbm, dst_hbm, w_hbm, out_hbm,
                     rows_v, src_v, dst_v, w_v, agg_sh, sem):
        cid = lax.axis_index("c")
        sid = lax.axis_index("s")
        wid = cid * 16 + sid

        zero16 = jnp.zeros((16,), jnp.float32)

        def zrow(e, carry):
            for p in range(F // 16):
                rows_v[e, pl.ds(p * 16, 16)] = zero16
            return carry

        lax.fori_loop(0, K, zrow, 0)
        for k in range(RT // K):
            pltpu.sync_copy(rows_v, agg_sh.at[pl.ds(sid * RT + k * K, K)])
        plsc.subcore_barrier()

        def chunk(c, carry):
            pltpu.sync_copy(src_hbm.at[wid, c], src_v)
            pltpu.sync_copy(dst_hbm.at[wid, c], dst_v)
            pltpu.sync_copy(w_hbm.at[wid, c], w_v)
            pltpu.async_copy(uhalf_hbm.at[src_v], rows_v, sem).wait()

            def mul(e, carry2):
                wv = w_v[e, :]
                for p in range(F // 16):
                    rows_v[e, pl.ds(p * 16, 16)] = (
                        rows_v[e, pl.ds(p * 16, 16)] * wv)
                return carry2

            lax.fori_loop(0, K, mul, 0)
            pltpu.sync_copy(rows_v, agg_sh.at[dst_v], add=True)
            return carry

        lax.fori_loop(0, C, chunk, 0)
        plsc.subcore_barrier()

        for k in range(RT // K):
            sl = pl.ds(sid * RT + k * K, K)
            pltpu.sync_copy(agg_sh.at[sl], out_hbm.at[cid, sl])

    return scatter_step


_GRID = 8
_BR = N_PAD // _GRID


def _pre_body(x_ref, we_ref, w1_ref, w2_ref, b_ref, uh_ref):
    h = jnp.dot(x_ref[...], we_ref[...], preferred_element_type=jnp.float32)
    t = jnp.tanh(jnp.dot(h, w1_ref[...], preferred_element_type=jnp.float32))
    b = jnp.dot(t, w2_ref[...], preferred_element_type=jnp.float32)
    b_ref[...] = b
    uh_ref[...] = -b


def _precompute(xp, W_enc, W_b1, W_b2):
    wspec = pl.BlockSpec((F, F), lambda i: (0, 0))
    rspec = pl.BlockSpec((_BR, F), lambda i: (i, 0))
    return pl.pallas_call(
        _pre_body,
        grid=(_GRID,),
        in_specs=[rspec, wspec, wspec, wspec],
        out_specs=[rspec, rspec],
        out_shape=[jax.ShapeDtypeStruct((N_PAD, F), jnp.float32),
                   jax.ShapeDtypeStruct((N_PAD, F), jnp.float32)],
    )(xp, W_enc, W_b1, W_b2)


def _upd_body(agg_ref, u_ref, uh_ref, b_ref, p_ref, un_ref, uhn_ref):
    agg = agg_ref[0] + agg_ref[1]
    c1 = p_ref[0, 0]
    cb = p_ref[0, 1]
    u = u_ref[...]
    un = c1 * uh_ref[...] + cb * agg - 2.0 * jnp.maximum(u, 0.0) + u
    un_ref[...] = un
    uhn_ref[...] = 2.0 * jnp.maximum(un, 0.0) - un - b_ref[...]


def _update(aggP, u, uh, b, params):
    rspec = pl.BlockSpec((_BR, F), lambda i: (i, 0))
    return pl.pallas_call(
        _upd_body,
        grid=(_GRID,),
        in_specs=[pl.BlockSpec((2, _BR, F), lambda i: (0, i, 0)),
                  rspec, rspec, rspec,
                  pl.BlockSpec((8, F), lambda i: (0, 0))],
        out_specs=[rspec, rspec],
        out_shape=[jax.ShapeDtypeStruct((N_PAD, F), jnp.float32),
                   jax.ShapeDtypeStruct((N_PAD, F), jnp.float32)],
    )(aggP, u, uh, b, params)


def _dec_body(u_ref, wd_ref, o_ref):
    o_ref[...] = jnp.dot(jnp.maximum(u_ref[...], 0.0), wd_ref[...],
                         preferred_element_type=jnp.float32)


def _decode(u, W_dec):
    rspec = pl.BlockSpec((_BR, F), lambda i: (i, 0))
    return pl.pallas_call(
        _dec_body,
        grid=(_GRID,),
        in_specs=[rspec, pl.BlockSpec((F, F), lambda i: (0, 0))],
        out_specs=rspec,
        out_shape=jax.ShapeDtypeStruct((N_PAD, F), jnp.float32),
    )(u, W_dec)


def kernel(x, edge_index, edge_weight, W_enc, W_b1, W_b2, W_dec, beta, gamma):
    n = x.shape[0]
    src = edge_index[0]
    dst = edge_index[1]
    e = src.shape[0]
    C = -(-e // (NW * K))
    pad = NW * C * K - e
    src_p = jnp.pad(src, (0, pad)).reshape(NW, C, K)
    dst_p = jnp.pad(dst, (0, pad)).reshape(NW, C, K)
    w_p = jnp.pad(edge_weight, (0, pad)).reshape(NW, C, K)
    w16 = jnp.broadcast_to(w_p[..., None], (NW, C, K, 16))

    xp = jnp.pad(x, ((0, N_PAD - n), (0, 0)))
    b, uh = _precompute(xp, W_enc, W_b1, W_b2)
    u = jnp.zeros_like(b)

    c1 = 2.0 / (1.0 + gamma)
    cb = c1 * beta
    params = jnp.zeros((8, F), jnp.float32).at[0, 0].set(c1).at[0, 1].set(cb)

    step = _make_scatter_step(C)
    for _ in range(ITERS):
        aggP = step(uh, src_p, dst_p, w16)
        u, uh = _update(aggP, u, uh, b, params)

    out = _decode(u, W_dec)
    return out[:n]
